# baseline (speedup 1.0000x reference)
import functools

import jax
import jax.numpy as jnp
from jax import lax
from jax.experimental import pallas as pl
from jax.experimental.pallas import tpu as pltpu
from jax.experimental.pallas import tpu_sc as plsc

N = 10000
E = 320000
D = 128
HD = D // 2
NC = 2
NS = 16
CH = 128
EPT = E // NS
NBUF = 4
CHUNKS = -(-EPT // CH)
CHUNKS = -(-CHUNKS // NBUF) * NBUF
EPT_PAD = CHUNKS * CH
RPT = 632
NROWS = NS * RPT


@functools.cache
def _sc_agg(with_deg):
    mesh = plsc.VectorSubcoreMesh(
        core_axis_name="c", subcore_axis_name="s", num_cores=NC, num_subcores=NS)

    out_type = [jax.ShapeDtypeStruct((NC, NROWS, HD), jnp.float32)]
    scratch = [
        pltpu.VMEM((CHUNKS, CH), jnp.int32),
        pltpu.VMEM((CHUNKS, CH), jnp.int32),
        [pltpu.VMEM((CH, HD), jnp.float32)] * NBUF,
        [pltpu.SemaphoreType.DMA] * NBUF,
        [pltpu.SemaphoreType.DMA] * NBUF,
        pltpu.VMEM_SHARED((NROWS, HD), jnp.float32),
    ]
    if with_deg:
        out_type.append(jax.ShapeDtypeStruct((NROWS, 16), jnp.float32))
        scratch.append(pltpu.VMEM((CH, 16), jnp.float32))
        scratch.append([pltpu.SemaphoreType.DMA] * NBUF)
        scratch.append(pltpu.VMEM_SHARED((NROWS, 16), jnp.float32))

    def body(table, srcr, dstr, zeros, zeros16, ones16,
             parts, degout, sidx, didx, rows, gsem, ssem, acc,
             onesv=None, dsem=None, degacc=None):
        c = lax.axis_index("c")
        s = lax.axis_index("s")
        r0 = s * RPT
        deg_on = with_deg

        pltpu.sync_copy(zeros, acc.at[pl.ds(r0, RPT)])
        if deg_on:
            pltpu.sync_copy(zeros16, degacc.at[pl.ds(r0, RPT)])
            pltpu.sync_copy(ones16, onesv)

        pltpu.sync_copy(srcr.at[c, s], sidx)
        pltpu.sync_copy(dstr.at[s], didx)
        plsc.subcore_barrier()

        def gather(chunk, b):
            pltpu.async_copy(table.at[sidx.at[chunk]], rows[b], gsem[b])

        def wait_gather(chunk, b):
            pltpu.make_async_copy(table.at[sidx.at[chunk]], rows[b],
                                  gsem[b]).wait()

        def scatter(chunk, b):
            pltpu.async_copy(rows[b], acc.at[didx.at[chunk]], ssem[b],
                             add=True)
            if deg_on:
                @pl.when(c == 0)
                def _():
                    pltpu.async_copy(onesv, degacc.at[didx.at[chunk]],
                                     dsem[b], add=True)

        def wait_scatter(chunk, b):
            pltpu.make_async_copy(rows[b], acc.at[didx.at[chunk]],
                                  ssem[b]).wait()
            if deg_on:
                @pl.when(c == 0)
                def _():
                    pltpu.make_async_copy(onesv, degacc.at[didx.at[chunk]],
                                          dsem[b]).wait()

        gather(0, 0)
        gather(1, 1)

        @pl.loop(0, CHUNKS, step=NBUF)
        def _(j):
            for b in range(NBUF):
                c = j + b
                b2 = (b + 2) % NBUF
                wait_gather(c, b)
                scatter(c, b)

                @pl.when(c >= 2)
                def _(c=c, b2=b2):
                    wait_scatter(c - 2, b2)

                @pl.when(c + 2 < CHUNKS)
                def _(c=c, b2=b2):
                    gather(c + 2, b2)

        for cc in (CHUNKS - 2, CHUNKS - 1):
            wait_scatter(cc, cc % NBUF)

        plsc.subcore_barrier()

        pltpu.sync_copy(acc.at[pl.ds(r0, RPT)], parts.at[c, pl.ds(r0, RPT)])
        if deg_on:
            @pl.when(c == 0)
            def _():
                pltpu.sync_copy(degacc.at[pl.ds(r0, RPT)],
                                degout.at[pl.ds(r0, RPT)])

    if with_deg:
        fn = body
    else:
        def fn(table, srcr, dstr, zeros, zeros16, ones16, parts,
               sidx, didx, rows, gsem, ssem, acc):
            return body(table, srcr, dstr, zeros, zeros16, ones16,
                        parts, None, sidx, didx, rows, gsem, ssem, acc)

    return pl.kernel(fn, out_type=out_type, mesh=mesh, scratch_types=scratch,
                     compiler_params=pltpu.CompilerParams(
                         use_tc_tiling_on_sc=False))


BLK = 1000


def _tc_layer(relu):
    def body(p_ref, dp_ref, h_ref, wl_ref, wr_ref, b_ref, o_ref):
        agg = jnp.concatenate((p_ref[0], p_ref[1]), axis=1)
        deg = dp_ref[:, 0:1]
        mean = agg / jnp.maximum(deg, 1.0)
        r = jnp.dot(mean, wl_ref[...], preferred_element_type=jnp.float32)
        r = r + jnp.dot(h_ref[...], wr_ref[...], preferred_element_type=jnp.float32)
        r = r + b_ref[...]
        if relu:
            r = jnp.maximum(r, 0.0)
        o_ref[...] = r

    return pl.pallas_call(
        body,
        grid=(N // BLK,),
        in_specs=[
            pl.BlockSpec((NC, BLK, HD), lambda i: (0, i, 0)),
            pl.BlockSpec((BLK, 16), lambda i: (i, 0)),
            pl.BlockSpec((BLK, D), lambda i: (i, 0)),
            pl.BlockSpec((D, D), lambda i: (0, 0)),
            pl.BlockSpec((D, D), lambda i: (0, 0)),
            pl.BlockSpec((1, D), lambda i: (0, 0)),
        ],
        out_specs=pl.BlockSpec((BLK, D), lambda i: (i, 0)),
        out_shape=jax.ShapeDtypeStruct((N, D), jnp.float32),
    )


_tc_relu = _tc_layer(True)
_tc_lin = _tc_layer(False)


def kernel(x, edge_index, W1_l, b1, W1_r, W2_l, b2, W2_r):
    src = edge_index[0]
    dst = edge_index[1]
    pad = EPT_PAD - EPT
    s2 = (src * 2).reshape(NS, EPT)
    s2 = jnp.pad(s2, ((0, 0), (0, pad)))
    srcr = jnp.stack((s2, s2 + 1)).reshape(NC, NS, CHUNKS, CH)
    dstr = jnp.pad(dst.reshape(NS, EPT), ((0, 0), (0, pad)),
                   constant_values=N).reshape(NS, CHUNKS, CH)

    zeros = jnp.zeros((RPT, HD), jnp.float32)
    zeros16 = jnp.zeros((RPT, 16), jnp.float32)
    ones16 = jnp.ones((CH, 16), jnp.float32)

    x2 = x.reshape(2 * N, HD)
    parts1, deg = _sc_agg(True)(x2, srcr, dstr, zeros, zeros16, ones16)
    h1 = _tc_relu(parts1, deg, x, W1_l.T, W1_r.T, b1.reshape(1, D))
    (parts2,) = _sc_agg(False)(h1.reshape(2 * N, HD), srcr, dstr,
                               zeros, zeros16, ones16)
    out = _tc_lin(parts2, deg, h1, W2_l.T, W2_r.T, b2.reshape(1, D))
    return out


# speedup vs baseline: 1.0424x; 1.0424x over previous
import functools

import jax
import jax.numpy as jnp
from jax import lax
from jax.experimental import pallas as pl
from jax.experimental.pallas import tpu as pltpu
from jax.experimental.pallas import tpu_sc as plsc

N = 10000
E = 320000
D = 128
HD = D // 2
NC = 2
NS = 16
CH = 128
EPT = E // NS
NBUF = 2
IB = 2
CH = IB * 128
CHUNKS = -(-EPT // CH)
CHUNKS = -(-CHUNKS // NBUF) * NBUF
EPT_PAD = CHUNKS * CH
RPT = 632
NROWS = NS * RPT


@functools.cache
def _sc_agg(with_deg):
    mesh = plsc.VectorSubcoreMesh(
        core_axis_name="c", subcore_axis_name="s", num_cores=NC, num_subcores=NS)

    out_type = [jax.ShapeDtypeStruct((NC, NROWS, HD), jnp.float32)]
    scratch = [
        pltpu.VMEM((CHUNKS, CH), jnp.int32),
        pltpu.VMEM((CHUNKS, CH), jnp.int32),
        [pltpu.VMEM((CH, HD), jnp.float32)] * NBUF,
        [pltpu.SemaphoreType.DMA] * NBUF,
        pltpu.VMEM_SHARED((NROWS, HD), jnp.float32),
    ]
    if with_deg:
        out_type.append(jax.ShapeDtypeStruct((NROWS, 16), jnp.float32))
        scratch.append(pltpu.VMEM((CH, 16), jnp.float32))
        scratch.append(pltpu.VMEM_SHARED((NROWS, 16), jnp.float32))

    def body(table, srcr, dstr, zeros, zeros16, ones16,
             parts, degout, sidx, didx, rows, gsem, acc,
             onesv=None, degacc=None):
        c = lax.axis_index("c")
        s = lax.axis_index("s")
        r0 = s * RPT
        deg_on = with_deg

        pltpu.sync_copy(zeros, acc.at[pl.ds(r0, RPT)])
        if deg_on:
            pltpu.sync_copy(zeros16, degacc.at[pl.ds(r0, RPT)])
            pltpu.sync_copy(ones16, onesv)

        pltpu.sync_copy(srcr.at[c, s], sidx)
        pltpu.sync_copy(dstr.at[s], didx)
        plsc.subcore_barrier()

        def gather(chunk, b):
            pltpu.async_copy(table.at[sidx.at[chunk]], rows[b], gsem[b])

        def wait_gather(chunk, b):
            pltpu.make_async_copy(table.at[sidx.at[chunk]], rows[b],
                                  gsem[b]).wait()

        def scatter(chunk, b):
            pltpu.sync_copy(rows[b], acc.at[didx.at[chunk]], add=True)
            if deg_on:
                @pl.when(c == 0)
                def _():
                    pltpu.sync_copy(onesv, degacc.at[didx.at[chunk]],
                                    add=True)

        gather(0, 0)

        @pl.loop(0, CHUNKS, step=NBUF)
        def _(j):
            gather(j + 1, 1)
            wait_gather(j, 0)
            scatter(j, 0)

            @pl.when(j + 2 < CHUNKS)
            def _():
                gather(j + 2, 0)

            wait_gather(j + 1, 1)
            scatter(j + 1, 1)

        plsc.subcore_barrier()

        pltpu.sync_copy(acc.at[pl.ds(r0, RPT)], parts.at[c, pl.ds(r0, RPT)])
        if deg_on:
            @pl.when(c == 0)
            def _():
                pltpu.sync_copy(degacc.at[pl.ds(r0, RPT)],
                                degout.at[pl.ds(r0, RPT)])

    if with_deg:
        fn = body
    else:
        def fn(table, srcr, dstr, zeros, zeros16, ones16, parts,
               sidx, didx, rows, gsem, acc):
            return body(table, srcr, dstr, zeros, zeros16, ones16,
                        parts, None, sidx, didx, rows, gsem, acc)

    return pl.kernel(fn, out_type=out_type, mesh=mesh, scratch_types=scratch,
                     compiler_params=pltpu.CompilerParams(
                         use_tc_tiling_on_sc=False))


BLK = 1000


def _tc_layer(relu):
    def body(p_ref, dp_ref, h_ref, wl_ref, wr_ref, b_ref, o_ref):
        agg = jnp.concatenate((p_ref[0], p_ref[1]), axis=1)
        deg = dp_ref[:, 0:1]
        mean = agg / jnp.maximum(deg, 1.0)
        r = jnp.dot(mean, wl_ref[...], preferred_element_type=jnp.float32)
        r = r + jnp.dot(h_ref[...], wr_ref[...], preferred_element_type=jnp.float32)
        r = r + b_ref[...]
        if relu:
            r = jnp.maximum(r, 0.0)
        o_ref[...] = r

    return pl.pallas_call(
        body,
        grid=(N // BLK,),
        in_specs=[
            pl.BlockSpec((NC, BLK, HD), lambda i: (0, i, 0)),
            pl.BlockSpec((BLK, 16), lambda i: (i, 0)),
            pl.BlockSpec((BLK, D), lambda i: (i, 0)),
            pl.BlockSpec((D, D), lambda i: (0, 0)),
            pl.BlockSpec((D, D), lambda i: (0, 0)),
            pl.BlockSpec((1, D), lambda i: (0, 0)),
        ],
        out_specs=pl.BlockSpec((BLK, D), lambda i: (i, 0)),
        out_shape=jax.ShapeDtypeStruct((N, D), jnp.float32),
    )


_tc_relu = _tc_layer(True)
_tc_lin = _tc_layer(False)


def kernel(x, edge_index, W1_l, b1, W1_r, W2_l, b2, W2_r):
    src = edge_index[0]
    dst = edge_index[1]
    pad = EPT_PAD - EPT
    s2 = (src * 2).reshape(NS, EPT)
    s2 = jnp.pad(s2, ((0, 0), (0, pad)))
    srcr = jnp.stack((s2, s2 + 1)).reshape(NC, NS, CHUNKS, CH)
    dstr = jnp.pad(dst.reshape(NS, EPT), ((0, 0), (0, pad)),
                   constant_values=N).reshape(NS, CHUNKS, CH)

    zeros = jnp.zeros((RPT, HD), jnp.float32)
    zeros16 = jnp.zeros((RPT, 16), jnp.float32)
    ones16 = jnp.ones((CH, 16), jnp.float32)

    x2 = x.reshape(2 * N, HD)
    parts1, deg = _sc_agg(True)(x2, srcr, dstr, zeros, zeros16, ones16)
    h1 = _tc_relu(parts1, deg, x, W1_l.T, W1_r.T, b1.reshape(1, D))
    (parts2,) = _sc_agg(False)(h1.reshape(2 * N, HD), srcr, dstr,
                               zeros, zeros16, ones16)
    out = _tc_lin(parts2, deg, h1, W2_l.T, W2_r.T, b2.reshape(1, D))
    return out


# speedup vs baseline: 1.6666x; 1.5989x over previous
import functools

import jax
import jax.numpy as jnp
from jax import lax
from jax.experimental import pallas as pl
from jax.experimental.pallas import tpu as pltpu
from jax.experimental.pallas import tpu_sc as plsc

N = 10000
E = 320000
D = 128
HD = D // 2
NC = 2
NS = 16
CH = 128
EPT = E // NS
NBUF = 2
CH = 64
CHUNKS = -(-EPT // CH)
CHUNKS = -(-CHUNKS // NBUF) * NBUF
EPT_PAD = CHUNKS * CH
RPT = 632
NROWS = NS * RPT


@functools.cache
def _sc_agg(with_deg):
    mesh = plsc.VectorSubcoreMesh(
        core_axis_name="c", subcore_axis_name="s", num_cores=NC, num_subcores=NS)

    out_type = [jax.ShapeDtypeStruct((NC, NROWS, HD), jnp.float32)]
    scratch = [
        pltpu.VMEM((CHUNKS, CH), jnp.int32),
        pltpu.VMEM((CHUNKS, CH), jnp.int32),
        [pltpu.VMEM((CH, HD), jnp.float32)] * NBUF,
        [pltpu.SemaphoreType.DMA] * NBUF,
        pltpu.VMEM_SHARED((NROWS, HD), jnp.float32),
    ]
    if with_deg:
        out_type.append(jax.ShapeDtypeStruct((NROWS, 16), jnp.float32))
        scratch.append(pltpu.VMEM((CH, 16), jnp.float32))
        scratch.append(pltpu.VMEM_SHARED((NROWS, 16), jnp.float32))

    def body(table, srcr, dstr, zeros, zeros16, ones16,
             parts, degout, sidx, didx, rows, gsem, acc,
             onesv=None, degacc=None):
        c = lax.axis_index("c")
        s = lax.axis_index("s")
        r0 = s * RPT
        deg_on = with_deg

        pltpu.sync_copy(zeros, acc.at[pl.ds(r0, RPT)])
        if deg_on:
            pltpu.sync_copy(zeros16, degacc.at[pl.ds(r0, RPT)])
            pltpu.sync_copy(ones16, onesv)

        pltpu.sync_copy(srcr.at[c, s], sidx)
        pltpu.sync_copy(dstr.at[s], didx)
        plsc.subcore_barrier()

        def gather(chunk, b):
            pltpu.async_copy(table.at[sidx.at[chunk]], rows[b], gsem[b])

        def wait_gather(chunk, b):
            pltpu.make_async_copy(table.at[sidx.at[chunk]], rows[b],
                                  gsem[b]).wait()

        def scatter(chunk, b):
            pltpu.sync_copy(rows[b], acc.at[didx.at[chunk]], add=True)
            if deg_on:
                @pl.when(c == 0)
                def _():
                    pltpu.sync_copy(onesv, degacc.at[didx.at[chunk]],
                                    add=True)

        gather(0, 0)

        @pl.loop(0, CHUNKS, step=NBUF)
        def _(j):
            gather(j + 1, 1)
            wait_gather(j, 0)
            scatter(j, 0)

            @pl.when(j + 2 < CHUNKS)
            def _():
                gather(j + 2, 0)

            wait_gather(j + 1, 1)
            scatter(j + 1, 1)

        plsc.subcore_barrier()

        pltpu.sync_copy(acc.at[pl.ds(r0, RPT)], parts.at[c, pl.ds(r0, RPT)])
        if deg_on:
            @pl.when(c == 0)
            def _():
                pltpu.sync_copy(degacc.at[pl.ds(r0, RPT)],
                                degout.at[pl.ds(r0, RPT)])

    if with_deg:
        fn = body
    else:
        def fn(table, srcr, dstr, zeros, zeros16, ones16, parts,
               sidx, didx, rows, gsem, acc):
            return body(table, srcr, dstr, zeros, zeros16, ones16,
                        parts, None, sidx, didx, rows, gsem, acc)

    return pl.kernel(fn, out_type=out_type, mesh=mesh, scratch_types=scratch,
                     compiler_params=pltpu.CompilerParams(
                         use_tc_tiling_on_sc=False))


BLK = 1000


def _tc_layer(relu):
    def body(p_ref, dp_ref, h_ref, wl_ref, wr_ref, b_ref, o_ref):
        agg = jnp.concatenate((p_ref[0], p_ref[1]), axis=1)
        deg = dp_ref[:, 0:1]
        mean = agg / jnp.maximum(deg, 1.0)
        r = jnp.dot(mean, wl_ref[...], preferred_element_type=jnp.float32)
        r = r + jnp.dot(h_ref[...], wr_ref[...], preferred_element_type=jnp.float32)
        r = r + b_ref[...]
        if relu:
            r = jnp.maximum(r, 0.0)
        o_ref[...] = r

    return pl.pallas_call(
        body,
        grid=(N // BLK,),
        in_specs=[
            pl.BlockSpec((NC, BLK, HD), lambda i: (0, i, 0)),
            pl.BlockSpec((BLK, 16), lambda i: (i, 0)),
            pl.BlockSpec((BLK, D), lambda i: (i, 0)),
            pl.BlockSpec((D, D), lambda i: (0, 0)),
            pl.BlockSpec((D, D), lambda i: (0, 0)),
            pl.BlockSpec((1, D), lambda i: (0, 0)),
        ],
        out_specs=pl.BlockSpec((BLK, D), lambda i: (i, 0)),
        out_shape=jax.ShapeDtypeStruct((N, D), jnp.float32),
    )


_tc_relu = _tc_layer(True)
_tc_lin = _tc_layer(False)


def kernel(x, edge_index, W1_l, b1, W1_r, W2_l, b2, W2_r):
    src = edge_index[0]
    dst = edge_index[1]
    pad = EPT_PAD - EPT
    s2 = (src * 2).reshape(NS, EPT)
    s2 = jnp.pad(s2, ((0, 0), (0, pad)))
    srcr = jnp.stack((s2, s2 + 1)).reshape(NC, NS, CHUNKS, CH)
    dstr = jnp.pad(dst.reshape(NS, EPT), ((0, 0), (0, pad)),
                   constant_values=N).reshape(NS, CHUNKS, CH)

    zeros = jnp.zeros((RPT, HD), jnp.float32)
    zeros16 = jnp.zeros((RPT, 16), jnp.float32)
    ones16 = jnp.ones((CH, 16), jnp.float32)

    x2 = x.reshape(2 * N, HD)
    parts1, deg = _sc_agg(True)(x2, srcr, dstr, zeros, zeros16, ones16)
    h1 = _tc_relu(parts1, deg, x, W1_l.T, W1_r.T, b1.reshape(1, D))
    (parts2,) = _sc_agg(False)(h1.reshape(2 * N, HD), srcr, dstr,
                               zeros, zeros16, ones16)
    out = _tc_lin(parts2, deg, h1, W2_l.T, W2_r.T, b2.reshape(1, D))
    return out
